# baseline (speedup 1.0000x reference)
import functools

import jax
import jax.numpy as jnp
from jax import lax
from jax.experimental import pallas as pl
from jax.experimental.pallas import tpu as pltpu
from jax.experimental.pallas import tpu_sc as plsc

B = 16384
F = 64
NC = 2
NS = 16
NW = NC * NS
BPW = B // NW
L = 16
BLK = 128


def _sc_pairmf(user, item_i, item_j, eu_t, ei_t):
    mesh = plsc.VectorSubcoreMesh(core_axis_name="c", subcore_axis_name="s")
    cp = pltpu.CompilerParams(
        needs_layout_passes=False,
        use_tc_tiling_on_sc=True,
        disable_bounds_checks=True,
    )
    out_type = (
        jax.ShapeDtypeStruct((B,), jnp.float32),
        jax.ShapeDtypeStruct((B,), jnp.float32),
    )
    blk = pltpu.VMEM((F, BLK), jnp.float32)

    @functools.partial(
        pl.kernel,
        out_type=out_type,
        mesh=mesh,
        compiler_params=cp,
        scratch_types=[
            pltpu.VMEM((BPW,), jnp.int32),
            pltpu.VMEM((BPW,), jnp.int32),
            pltpu.VMEM((BPW,), jnp.int32),
            [[blk, blk], [blk, blk], [blk, blk]],
            [[blk, blk], [blk, blk], [blk, blk]],
            pltpu.VMEM((BPW,), jnp.float32),
            pltpu.VMEM((BPW,), jnp.float32),
            pltpu.SemaphoreType.DMA,
            pltpu.SemaphoreType.DMA,
        ],
    )
    def k(user_hbm, ii_hbm, ij_hbm, eu_hbm, ei_hbm, oi_hbm, oj_hbm,
          idx_u, idx_i, idx_j, bufs0, bufs1, oi_v, oj_v, sem0, sem1):
        wid = lax.axis_index("s") * NC + lax.axis_index("c")
        base = wid * BPW

        pltpu.sync_copy(user_hbm.at[pl.ds(base, BPW)], idx_u)
        pltpu.sync_copy(ii_hbm.at[pl.ds(base, BPW)], idx_i)
        pltpu.sync_copy(ij_hbm.at[pl.ds(base, BPW)], idx_j)

        lane = lax.iota(jnp.int32, L)
        m15 = lane == (L - 1)
        zero16 = jnp.zeros((L,), jnp.int32)
        bufs = (bufs0, bufs1)
        sems = (sem0, sem1)
        tables = (eu_hbm, ei_hbm, ei_hbm)
        idxs = (idx_u, idx_i, idx_j)

        def extract(idx_v, r):
            chunk = idx_v[pl.ds((r // L) * L, L)]
            return jnp.sum(jnp.where(lane == (r % L), chunk, zero16))

        def issue(row, phase, slot, sem):
            for t in range(3):
                v = extract(idxs[t], row)
                off = pl.multiple_of(
                    lax.shift_right_logical(v, 7) * BLK, BLK)
                pltpu.async_copy(
                    tables[t].at[:, pl.ds(off, BLK)],
                    bufs[phase][t][slot], sem)

        def drain(phase, slot, sem):
            for t in range(3):
                pltpu.make_async_copy(
                    tables[t].at[:, pl.ds(0, BLK)],
                    bufs[phase][t][slot], sem).wait()

        def compute(row, phase, slot):
            cs = [lax.bitwise_and(extract(idxs[t], row), BLK - 1)
                  for t in range(3)]
            cvecs = [jnp.full((L,), c, jnp.int32) for c in cs]
            ub, ib, jb = (bufs[phase][t][slot] for t in range(3))
            acc_i = jnp.zeros((L,), jnp.float32)
            acc_j = jnp.zeros((L,), jnp.float32)
            for g in range(4):
                fvec = lane + g * L
                u = plsc.load_gather(ub, [fvec, cvecs[0]])
                acc_i = acc_i + u * plsc.load_gather(ib, [fvec, cvecs[1]])
                acc_j = acc_j + u * plsc.load_gather(jb, [fvec, cvecs[2]])
            rvec = jnp.full((L,), row, jnp.int32)
            plsc.store_scatter(oi_v, [rvec], plsc.cumsum(acc_i), mask=m15)
            plsc.store_scatter(oj_v, [rvec], plsc.cumsum(acc_j), mask=m15)

        for p in range(2):
            for s in range(2):
                issue(2 * p + s, p, s, sems[p])

        @pl.loop(0, BPW // 4)
        def _(kk):
            r0 = kk * 4
            for p in range(2):
                for s in range(2):
                    drain(p, s, sems[p])
                for s in range(2):
                    compute(r0 + 2 * p + s, p, s)

                @pl.when(kk < BPW // 4 - 1)
                def _():
                    for s in range(2):
                        issue(r0 + 4 + 2 * p + s, p, s, sems[p])

        pltpu.sync_copy(oi_v, oi_hbm.at[pl.ds(base, BPW)])
        pltpu.sync_copy(oj_v, oj_hbm.at[pl.ds(base, BPW)])

    return k(user, item_i, item_j, eu_t, ei_t)


def kernel(user, item_i, item_j, embed_user, embed_item):
    user = user.astype(jnp.int32)
    item_i = item_i.astype(jnp.int32)
    item_j = item_j.astype(jnp.int32)
    pi, pj = _sc_pairmf(user, item_i, item_j, embed_user.T, embed_item.T)
    su, pu = jax.lax.sort([user, jax.lax.iota(jnp.int32, B)], num_keys=1)
    items = jnp.concatenate([item_i, item_j])
    si, pit = jax.lax.sort([items, jax.lax.iota(jnp.int32, 2 * B)], num_keys=1)
    bu = su >> 7
    isnew = jnp.concatenate([jnp.ones((1,), jnp.int32),
                             (jnp.diff(bu) != 0).astype(jnp.int32)])
    slots = jnp.cumsum(isnew)
    rank_u = jnp.zeros((B,), jnp.int32).at[pu].set(jax.lax.iota(jnp.int32, B))
    rank_i = jnp.zeros((2 * B,), jnp.int32).at[pit].set(
        jax.lax.iota(jnp.int32, 2 * B))
    dead = (slots[-1] + rank_u[0] + rank_i[0] + si[0]).astype(jnp.float32)
    return pi + 0.0 * dead, pj


# speedup vs baseline: 1.5476x; 1.5476x over previous
import functools

import jax
import jax.numpy as jnp
from jax import lax
from jax.experimental import pallas as pl
from jax.experimental.pallas import tpu as pltpu
from jax.experimental.pallas import tpu_sc as plsc

B = 16384
F = 64
NC = 2
NS = 16
NW = NC * NS
L = 16
BLK = 128
DEPTH = 8
CHUNK = 64

_i32 = jnp.int32


def _band(a, b):
    return lax.bitwise_and(a, _i32(b))


def _gather_kernel(su, si, slotinfo_u, slotinfo_i, dlist_u, dlist_i,
                   eu_t, ei_t):
    mesh = plsc.VectorSubcoreMesh(core_axis_name="c", subcore_axis_name="s")
    cp = pltpu.CompilerParams(
        needs_layout_passes=False,
        use_tc_tiling_on_sc=True,
        disable_bounds_checks=True,
    )
    out_type = (
        jax.ShapeDtypeStruct((B * F,), jnp.float32),
        jax.ShapeDtypeStruct((2 * B * F,), jnp.float32),
    )

    @functools.partial(
        pl.kernel,
        out_type=out_type,
        mesh=mesh,
        compiler_params=cp,
        scratch_types=[
            pltpu.VMEM((1024,), _i32),
            pltpu.VMEM((1024,), _i32),
            pltpu.VMEM((1040,), _i32),
            pltpu.VMEM((DEPTH, F, BLK), jnp.float32),
            pltpu.VMEM((2, CHUNK * F), jnp.float32),
            pltpu.SemaphoreType.DMA,
            pltpu.SemaphoreType.DMA,
        ],
    )
    def k(su_hbm, si_hbm, slu_hbm, sli_hbm, dlu_hbm, dli_hbm,
          eu_hbm, ei_hbm, gu_hbm, gi_hbm,
          sidx_v, slot_v, dl_v, arena, staging, sem, sem_out):
        wid = lax.axis_index("s") * NC + lax.axis_index("c")
        lane = lax.iota(_i32, L)
        zero16 = jnp.zeros((L,), _i32)

        def extract(vec_ref, r):
            off = pl.multiple_of((r // L) * L, L)
            chunk = vec_ref[pl.ds(off, L)]
            return jnp.sum(jnp.where(lane == (r % L), chunk, zero16))

        def run_stream(sidx_hbm, slotinfo_hbm, dlist_hbm, tbl, gout, n):
            base = wid * n
            pltpu.sync_copy(sidx_hbm.at[pl.ds(base, n)],
                            sidx_v.at[pl.ds(0, n)])
            pltpu.sync_copy(slotinfo_hbm.at[pl.ds(base, n)],
                            slot_v.at[pl.ds(0, n)])
            s0 = lax.shift_right_logical(extract(slot_v, 0), 1)
            slast = lax.shift_right_logical(extract(slot_v, n - 1), 1)
            s0a = pl.multiple_of(
                lax.shift_left(lax.shift_right_logical(s0, 3), 3), 8)
            pltpu.sync_copy(dlist_hbm.at[pl.ds(s0a, n + 8)],
                            dl_v.at[pl.ds(0, n + 8)])

            def issue(s):
                b = extract(dl_v, s - s0a)
                off = pl.multiple_of(b * BLK, BLK)
                pltpu.async_copy(tbl.at[:, pl.ds(off, BLK)],
                                 arena.at[_band(s, DEPTH - 1)], sem)

            for kk in range(DEPTH - 1):
                @pl.when(s0 + kk <= slast)
                def _():
                    issue(s0 + kk)

            @pl.loop(0, n)
            def _(r):
                rr = _band(r, CHUNK - 1)
                ck = lax.shift_right_logical(r, 6)
                p = _band(ck, 1)

                @pl.when(jnp.logical_and(rr == 0, ck >= 2))
                def _():
                    pltpu.make_async_copy(
                        gout.at[pl.ds(0, CHUNK * F)], staging.at[p],
                        sem_out).wait()

                info = extract(slot_v, r)
                s = lax.shift_right_logical(info, 1)
                isn = _band(info, 1)
                d = _band(s, DEPTH - 1)

                @pl.when(jnp.logical_or(isn == 1, r == 0))
                def _():
                    pltpu.make_async_copy(
                        tbl.at[:, pl.ds(0, BLK)], arena.at[d], sem).wait()
                    s2 = s + DEPTH - 1

                    @pl.when(s2 <= slast)
                    def _():
                        issue(s2)

                c = _band(extract(sidx_v, r), BLK - 1)
                csp = jnp.full((L,), c, _i32)
                dsp = jnp.full((L,), d, _i32)
                for g in range(4):
                    v = plsc.load_gather(arena, [dsp, lane + g * L, csp])
                    soff = pl.multiple_of(rr * F + g * L, L)
                    staging[p, pl.ds(soff, L)] = v

                @pl.when(rr == CHUNK - 1)
                def _():
                    goff = pl.multiple_of(
                        (base + r - (CHUNK - 1)) * F, CHUNK * F)
                    pltpu.async_copy(
                        staging.at[p], gout.at[pl.ds(goff, CHUNK * F)],
                        sem_out)

            for _unused in range(2):
                pltpu.make_async_copy(
                    gout.at[pl.ds(0, CHUNK * F)], staging.at[0],
                    sem_out).wait()

        run_stream(su_hbm, slu_hbm, dlu_hbm, eu_hbm, gu_hbm, B // NW)
        run_stream(si_hbm, sli_hbm, dli_hbm, ei_hbm, gi_hbm, 2 * B // NW)

    return k(su, si, slotinfo_u, slotinfo_i, dlist_u, dlist_i, eu_t, ei_t)


def _dot_kernel(rk_u, rk_i, rk_j, gu, gi):
    mesh = plsc.VectorSubcoreMesh(core_axis_name="c", subcore_axis_name="s")
    cp = pltpu.CompilerParams(
        needs_layout_passes=False, use_tc_tiling_on_sc=False)
    BPW = B // NW
    out_type = (
        jax.ShapeDtypeStruct((B,), jnp.float32),
        jax.ShapeDtypeStruct((B,), jnp.float32),
    )

    @functools.partial(
        pl.kernel,
        out_type=out_type,
        mesh=mesh,
        compiler_params=cp,
        scratch_types=[
            pltpu.VMEM((BPW,), _i32),
            pltpu.VMEM((BPW,), _i32),
            pltpu.VMEM((BPW,), _i32),
            pltpu.VMEM((BPW, F), jnp.float32),
            pltpu.VMEM((BPW, F), jnp.float32),
            pltpu.VMEM((BPW, F), jnp.float32),
            pltpu.VMEM((BPW,), jnp.float32),
            pltpu.VMEM((BPW,), jnp.float32),
            pltpu.SemaphoreType.DMA,
            pltpu.SemaphoreType.DMA,
            pltpu.SemaphoreType.DMA,
        ],
    )
    def k(rku_hbm, rki_hbm, rkj_hbm, gu_hbm, gi_hbm, oi_hbm, oj_hbm,
          uidx, iidx, jidx, urows, irows, jrows, oi_v, oj_v, su_, si_, sj_):
        wid = lax.axis_index("s") * NC + lax.axis_index("c")
        base = wid * BPW

        pltpu.sync_copy(rku_hbm.at[pl.ds(base, BPW)], uidx)
        pltpu.sync_copy(rki_hbm.at[pl.ds(base, BPW)], iidx)
        pltpu.sync_copy(rkj_hbm.at[pl.ds(base, BPW)], jidx)

        cu = pltpu.async_copy(gu_hbm.at[uidx], urows, su_)
        ci = pltpu.async_copy(gi_hbm.at[iidx], irows, si_)
        cj = pltpu.async_copy(gi_hbm.at[jidx], jrows, sj_)
        cu.wait()
        ci.wait()
        cj.wait()

        lane = lax.iota(_i32, L)
        m15 = lane == (L - 1)

        @pl.loop(0, BPW)
        def _(r):
            u0 = urows[r, pl.ds(0, L)]
            u1 = urows[r, pl.ds(L, L)]
            u2 = urows[r, pl.ds(2 * L, L)]
            u3 = urows[r, pl.ds(3 * L, L)]
            a0 = irows[r, pl.ds(0, L)]
            a1 = irows[r, pl.ds(L, L)]
            a2 = irows[r, pl.ds(2 * L, L)]
            a3 = irows[r, pl.ds(3 * L, L)]
            b0 = jrows[r, pl.ds(0, L)]
            b1 = jrows[r, pl.ds(L, L)]
            b2 = jrows[r, pl.ds(2 * L, L)]
            b3 = jrows[r, pl.ds(3 * L, L)]
            acc_i = u0 * a0 + u1 * a1 + u2 * a2 + u3 * a3
            acc_j = u0 * b0 + u1 * b1 + u2 * b2 + u3 * b3
            rvec = jnp.full((L,), r, _i32)
            plsc.store_scatter(oi_v, [rvec], plsc.cumsum(acc_i), mask=m15)
            plsc.store_scatter(oj_v, [rvec], plsc.cumsum(acc_j), mask=m15)

        pltpu.sync_copy(oi_v, oi_hbm.at[pl.ds(base, BPW)])
        pltpu.sync_copy(oj_v, oj_hbm.at[pl.ds(base, BPW)])

    return k(rk_u, rk_i, rk_j, gu, gi)


def _stream_meta(sorted_idx, pad):
    blocks = lax.shift_right_logical(sorted_idx, 7)
    n = sorted_idx.shape[0]
    isnew = jnp.concatenate(
        [jnp.ones((1,), _i32), (jnp.diff(blocks) != 0).astype(_i32)])
    slot = jnp.cumsum(isnew, dtype=_i32) - 1
    slotinfo = slot * 2 + isnew
    dlist = jnp.zeros((n + pad,), _i32).at[slot].set(blocks)
    return slotinfo, dlist


def kernel(user, item_i, item_j, embed_user, embed_item):
    user = user.astype(_i32)
    item_i = item_i.astype(_i32)
    item_j = item_j.astype(_i32)

    iota_b = lax.iota(_i32, B)
    iota_2b = lax.iota(_i32, 2 * B)
    su, pu = lax.sort([user, iota_b], num_keys=1)
    items = jnp.concatenate([item_i, item_j])
    si, pit = lax.sort([items, iota_2b], num_keys=1)
    rank_u = jnp.zeros((B,), _i32).at[pu].set(iota_b)
    rank_it = jnp.zeros((2 * B,), _i32).at[pit].set(iota_2b)

    slotinfo_u, dlist_u = _stream_meta(su, 16)
    slotinfo_i, dlist_i = _stream_meta(si, 16)

    gu_flat, gi_flat = _gather_kernel(
        su, si, slotinfo_u, slotinfo_i, dlist_u, dlist_i,
        embed_user.T, embed_item.T)
    gu = gu_flat.reshape(B, F)
    gi = gi_flat.reshape(2 * B, F)

    return _dot_kernel(rank_u, rank_it[:B], rank_it[B:], gu, gi)


# speedup vs baseline: 1.6072x; 1.0385x over previous
import functools

import jax
import jax.numpy as jnp
from jax import lax
from jax.experimental import pallas as pl
from jax.experimental.pallas import tpu as pltpu
from jax.experimental.pallas import tpu_sc as plsc

B = 16384
F = 64
NC = 2
NS = 16
NW = NC * NS
L = 16
BLK = 128
DEPTH = 8
CHUNK = 64

_i32 = jnp.int32


def _band(a, b):
    return lax.bitwise_and(a, _i32(b))


def _gather_kernel(sidx, slotinfo, dlist, tbl_t, n_total):
    mesh = plsc.VectorSubcoreMesh(core_axis_name="c", subcore_axis_name="s")
    cp = pltpu.CompilerParams(
        needs_layout_passes=False,
        use_tc_tiling_on_sc=True,
        disable_bounds_checks=True,
    )
    n = n_total // NW
    out_type = jax.ShapeDtypeStruct((n_total * F,), jnp.float32)

    @functools.partial(
        pl.kernel,
        out_type=out_type,
        mesh=mesh,
        compiler_params=cp,
        scratch_types=[
            pltpu.VMEM((n,), _i32),
            pltpu.VMEM((n,), _i32),
            pltpu.VMEM((n + 16,), _i32),
            pltpu.VMEM((DEPTH, F, BLK), jnp.float32),
            pltpu.VMEM((2, CHUNK * F), jnp.float32),
            pltpu.SemaphoreType.DMA,
            pltpu.SemaphoreType.DMA,
        ],
    )
    def k(sidx_hbm, slotinfo_hbm, dlist_hbm, tbl, gout,
          sidx_v, slot_v, dl_v, arena, staging, sem, sem_out):
        wid = lax.axis_index("s") * NC + lax.axis_index("c")
        lane = lax.iota(_i32, L)
        zero16 = jnp.zeros((L,), _i32)
        base = wid * n

        def extract(vec_ref, r):
            off = pl.multiple_of((r // L) * L, L)
            chunk = vec_ref[pl.ds(off, L)]
            return jnp.sum(jnp.where(lane == (r % L), chunk, zero16))

        pltpu.sync_copy(sidx_hbm.at[pl.ds(base, n)], sidx_v)
        pltpu.sync_copy(slotinfo_hbm.at[pl.ds(base, n)], slot_v)
        s0 = lax.shift_right_logical(extract(slot_v, 0), 1)
        slast = lax.shift_right_logical(extract(slot_v, n - 1), 1)
        s0a = pl.multiple_of(
            lax.shift_left(lax.shift_right_logical(s0, 3), 3), 8)
        pltpu.sync_copy(dlist_hbm.at[pl.ds(s0a, n + 8)],
                        dl_v.at[pl.ds(0, n + 8)])

        def issue(s):
            b = extract(dl_v, s - s0a)
            off = pl.multiple_of(b * BLK, BLK)
            pltpu.async_copy(tbl.at[:, pl.ds(off, BLK)],
                             arena.at[_band(s, DEPTH - 1)], sem)

        for kk in range(DEPTH - 1):
            @pl.when(s0 + kk <= slast)
            def _():
                issue(s0 + kk)

        @pl.loop(0, n)
        def _(r):
            rr = _band(r, CHUNK - 1)
            ck = lax.shift_right_logical(r, 6)
            p = _band(ck, 1)

            @pl.when(jnp.logical_and(rr == 0, ck >= 2))
            def _():
                pltpu.make_async_copy(
                    gout.at[pl.ds(0, CHUNK * F)], staging.at[p],
                    sem_out).wait()

            info = extract(slot_v, r)
            s = lax.shift_right_logical(info, 1)
            isn = _band(info, 1)
            d = _band(s, DEPTH - 1)

            @pl.when(jnp.logical_or(isn == 1, r == 0))
            def _():
                pltpu.make_async_copy(
                    tbl.at[:, pl.ds(0, BLK)], arena.at[d], sem).wait()
                s2 = s + DEPTH - 1

                @pl.when(s2 <= slast)
                def _():
                    issue(s2)

            c = _band(extract(sidx_v, r), BLK - 1)
            csp = jnp.full((L,), c, _i32)
            dsp = jnp.full((L,), d, _i32)
            for g in range(4):
                v = plsc.load_gather(arena, [dsp, lane + g * L, csp])
                soff = pl.multiple_of(rr * F + g * L, L)
                staging[p, pl.ds(soff, L)] = v

            @pl.when(rr == CHUNK - 1)
            def _():
                goff = pl.multiple_of(
                    (base + r - (CHUNK - 1)) * F, CHUNK * F)
                pltpu.async_copy(
                    staging.at[p], gout.at[pl.ds(goff, CHUNK * F)],
                    sem_out)

        for _unused in range(2):
            pltpu.make_async_copy(
                gout.at[pl.ds(0, CHUNK * F)], staging.at[0],
                sem_out).wait()

    return k(sidx, slotinfo, dlist, tbl_t)


def _dot_kernel(rk_u, rk_i, rk_j, gu, gi):
    mesh = plsc.VectorSubcoreMesh(core_axis_name="c", subcore_axis_name="s")
    cp = pltpu.CompilerParams(
        needs_layout_passes=False, use_tc_tiling_on_sc=False)
    BPW = B // NW
    out_type = (
        jax.ShapeDtypeStruct((B,), jnp.float32),
        jax.ShapeDtypeStruct((B,), jnp.float32),
    )

    @functools.partial(
        pl.kernel,
        out_type=out_type,
        mesh=mesh,
        compiler_params=cp,
        scratch_types=[
            pltpu.VMEM((BPW,), _i32),
            pltpu.VMEM((BPW,), _i32),
            pltpu.VMEM((BPW,), _i32),
            pltpu.VMEM((BPW, F), jnp.float32),
            pltpu.VMEM((BPW, F), jnp.float32),
            pltpu.VMEM((BPW, F), jnp.float32),
            pltpu.VMEM((BPW,), jnp.float32),
            pltpu.VMEM((BPW,), jnp.float32),
            pltpu.SemaphoreType.DMA,
            pltpu.SemaphoreType.DMA,
            pltpu.SemaphoreType.DMA,
        ],
    )
    def k(rku_hbm, rki_hbm, rkj_hbm, gu_hbm, gi_hbm, oi_hbm, oj_hbm,
          uidx, iidx, jidx, urows, irows, jrows, oi_v, oj_v, su_, si_, sj_):
        wid = lax.axis_index("s") * NC + lax.axis_index("c")
        base = wid * BPW

        pltpu.sync_copy(rku_hbm.at[pl.ds(base, BPW)], uidx)
        pltpu.sync_copy(rki_hbm.at[pl.ds(base, BPW)], iidx)
        pltpu.sync_copy(rkj_hbm.at[pl.ds(base, BPW)], jidx)

        cu = pltpu.async_copy(gu_hbm.at[uidx], urows, su_)
        ci = pltpu.async_copy(gi_hbm.at[iidx], irows, si_)
        cj = pltpu.async_copy(gi_hbm.at[jidx], jrows, sj_)
        cu.wait()
        ci.wait()
        cj.wait()

        lane = lax.iota(_i32, L)
        m15 = lane == (L - 1)

        @pl.loop(0, BPW)
        def _(r):
            u0 = urows[r, pl.ds(0, L)]
            u1 = urows[r, pl.ds(L, L)]
            u2 = urows[r, pl.ds(2 * L, L)]
            u3 = urows[r, pl.ds(3 * L, L)]
            a0 = irows[r, pl.ds(0, L)]
            a1 = irows[r, pl.ds(L, L)]
            a2 = irows[r, pl.ds(2 * L, L)]
            a3 = irows[r, pl.ds(3 * L, L)]
            b0 = jrows[r, pl.ds(0, L)]
            b1 = jrows[r, pl.ds(L, L)]
            b2 = jrows[r, pl.ds(2 * L, L)]
            b3 = jrows[r, pl.ds(3 * L, L)]
            acc_i = u0 * a0 + u1 * a1 + u2 * a2 + u3 * a3
            acc_j = u0 * b0 + u1 * b1 + u2 * b2 + u3 * b3
            rvec = jnp.full((L,), r, _i32)
            plsc.store_scatter(oi_v, [rvec], plsc.cumsum(acc_i), mask=m15)
            plsc.store_scatter(oj_v, [rvec], plsc.cumsum(acc_j), mask=m15)

        pltpu.sync_copy(oi_v, oi_hbm.at[pl.ds(base, BPW)])
        pltpu.sync_copy(oj_v, oj_hbm.at[pl.ds(base, BPW)])

    return k(rk_u, rk_i, rk_j, gu, gi)


def _stream_meta(sorted_idx, pad):
    blocks = lax.shift_right_logical(sorted_idx, 7)
    n = sorted_idx.shape[0]
    isnew = jnp.concatenate(
        [jnp.ones((1,), _i32), (jnp.diff(blocks) != 0).astype(_i32)])
    slot = jnp.cumsum(isnew, dtype=_i32) - 1
    slotinfo = slot * 2 + isnew
    dlist = jnp.zeros((n + pad,), _i32).at[slot].set(blocks)
    return slotinfo, dlist


def kernel(user, item_i, item_j, embed_user, embed_item):
    user = user.astype(_i32)
    item_i = item_i.astype(_i32)
    item_j = item_j.astype(_i32)

    iota_b = lax.iota(_i32, B)
    su, pu = lax.sort([user, iota_b], num_keys=1)
    rank_u = jnp.zeros((B,), _i32).at[pu].set(iota_b)
    slotinfo_u, dlist_u = _stream_meta(su, 16)
    gu_flat = _gather_kernel(su, slotinfo_u, dlist_u, embed_user.T, B)

    iota_2b = lax.iota(_i32, 2 * B)
    items = jnp.concatenate([item_i, item_j])
    si, pit = lax.sort([items, iota_2b], num_keys=1)
    rank_it = jnp.zeros((2 * B,), _i32).at[pit].set(iota_2b)
    slotinfo_i, dlist_i = _stream_meta(si, 16)
    gi_flat = _gather_kernel(si, slotinfo_i, dlist_i, embed_item.T, 2 * B)

    gu = gu_flat.reshape(B, F)
    gi = gi_flat.reshape(2 * B, F)

    return _dot_kernel(rank_u, rank_it[:B], rank_it[B:], gu, gi)
